# baseline (speedup 1.0000x reference)
import functools
import math

import jax
import jax.numpy as jnp
from jax import lax
from jax.experimental import pallas as pl
from jax.experimental.pallas import tpu as pltpu
from jax.experimental.pallas import tpu_sc as plsc

EMB = 64
LANES = 16
IDXW = 128
K = 2
C = K * IDXW
NC = 2
NS = 16
NW = NC * NS
SCALE = math.sqrt(EMB)


def _emb_body(idx_hbm, table_hbm, out_hbm,
              idx_v, rows0, rows1, sg0, sg1, sw0, sw1):
    wid = lax.axis_index("s") * NC + lax.axis_index("c")
    tok_w = idx_hbm.shape[0] // NW
    chunks = tok_w // C
    tok0 = pl.multiple_of(wid * tok_w, 1024)

    rows = (rows0, rows1)
    sg = (sg0, sg1)
    sw = (sw0, sw1)

    pltpu.sync_copy(idx_hbm.at[pl.ds(tok0, tok_w)], idx_v)

    def fire_gathers(ch, buf):
        for j in range(K):
            pltpu.async_copy(
                table_hbm.at[idx_v.at[pl.ds((ch * K + j) * IDXW, IDXW)]],
                rows[buf].at[pl.ds(j * IDXW, IDXW)],
                sg[buf],
            )

    def wait_gathers(buf):
        for _ in range(K):
            pltpu.make_async_copy(out_hbm.at[pl.ds(0, IDXW)],
                                  rows[buf].at[pl.ds(0, IDXW)], sg[buf]).wait()

    def fire_wb(ch, buf):
        base = pl.multiple_of((tok0 + ch * C) // 2, C // 2)
        pltpu.async_copy(rows[buf].at[pl.ds(0, C // 2)],
                         out_hbm.at[pl.ds(base, C // 2)],
                         sw[buf])

    def wait_wb(buf):
        pltpu.make_async_copy(rows[buf].at[pl.ds(0, C // 2)],
                              out_hbm.at[pl.ds(0, C // 2)], sw[buf]).wait()

    def pack(buf):
        r = rows[buf]

        def body(m, carry):
            for c in range(EMB // LANES):
                sl = pl.ds(c * LANES, LANES)
                r[m, sl] = r[m, sl] * SCALE
            for c in range(EMB // LANES):
                v = r[C // 2 + m, pl.ds(c * LANES, LANES)] * SCALE
                r[m, pl.ds(EMB + c * LANES, LANES)] = v
            return carry

        lax.fori_loop(0, C // 2, body, 0)

    def run_chunk(ch, buf, fire_next, wait_prev_wb):
        if wait_prev_wb:
            wait_wb(1 - buf)
        if fire_next:
            fire_gathers(ch + 1, 1 - buf)
        wait_gathers(buf)
        pack(buf)
        fire_wb(ch, buf)

    fire_gathers(0, 0)
    run_chunk(0, 0, fire_next=True, wait_prev_wb=False)

    def pair_body(i, carry):
        ch = 1 + 2 * i
        run_chunk(ch, 1, fire_next=True, wait_prev_wb=True)
        run_chunk(ch + 1, 0, fire_next=True, wait_prev_wb=True)
        return carry

    lax.fori_loop(0, (chunks - 2) // 2, pair_body, 0)

    run_chunk(chunks - 1, 1, fire_next=False, wait_prev_wb=True)
    wait_wb(1)


def kernel(tokens, table):
    b, t = tokens.shape
    n = b * t
    tp = tokens.T.astype(jnp.int32).reshape(n // C, C // 2, 2)
    idx = tp.transpose(0, 2, 1).reshape(n)
    table_pad = jnp.pad(table, ((0, 0), (0, 2 * EMB - table.shape[1])))

    mesh = plsc.VectorSubcoreMesh(core_axis_name="c", subcore_axis_name="s")
    run = functools.partial(
        pl.kernel,
        mesh=mesh,
        compiler_params=pltpu.CompilerParams(use_tc_tiling_on_sc=True),
        out_type=jax.ShapeDtypeStruct((n // 2, 2 * EMB), jnp.float32),
        scratch_types=[
            pltpu.VMEM((n // NW,), jnp.int32),
            pltpu.VMEM((C, 2 * EMB), jnp.float32),
            pltpu.VMEM((C, 2 * EMB), jnp.float32),
            pltpu.SemaphoreType.DMA,
            pltpu.SemaphoreType.DMA,
            pltpu.SemaphoreType.DMA,
            pltpu.SemaphoreType.DMA,
        ],
    )(_emb_body)
    out = run(idx, table_pad)
    return out.reshape(t, b, EMB).transpose(1, 0, 2)


# speedup vs baseline: 1.1331x; 1.1331x over previous
import functools
import math

import jax
import jax.numpy as jnp
from jax import lax
from jax.experimental import pallas as pl
from jax.experimental.pallas import tpu as pltpu
from jax.experimental.pallas import tpu_sc as plsc

EMB = 64
LANES = 16
IDXW = 128
K = 2
C = K * IDXW
NC = 2
NS = 16
NW = NC * NS
SCALE = math.sqrt(EMB)


def _emb_body(tok_hbm, table_hbm, out_hbm,
              tok_v, gidx0, gidx1, rows0, rows1, sg0, sg1, sw0, sw1):
    wid = lax.axis_index("s") * NC + lax.axis_index("c")
    n = tok_hbm.shape[0]
    tok_w = n // NW
    chunks = tok_w // C
    nb = out_hbm.shape[1]
    tok0 = pl.multiple_of(wid * tok_w, 1024)

    rows = (rows0, rows1)
    gidx = (gidx0, gidx1)
    sg = (sg0, sg1)
    sw = (sw0, sw1)

    pltpu.sync_copy(tok_hbm.at[pl.ds(tok0, tok_w)], tok_v)

    def fire_gathers(ch, buf):
        base = pl.multiple_of(ch * C, C)
        g = gidx[buf]
        for i in range(C // LANES):
            g[pl.ds(i * LANES, LANES)] = jnp.right_shift(
                tok_v[pl.ds(base + i * LANES, LANES)], 1)
        for j in range(K):
            pltpu.async_copy(
                table_hbm.at[g.at[pl.ds(j * IDXW, IDXW)]],
                rows[buf].at[pl.ds(j * IDXW, IDXW)],
                sg[buf],
            )

    def wait_gathers(buf):
        for _ in range(K):
            pltpu.make_async_copy(table_hbm.at[pl.ds(0, IDXW)],
                                  rows[buf].at[pl.ds(0, IDXW)], sg[buf]).wait()

    def out_slab(ch, buf):
        base = tok0 + ch * C
        tt = base // nb
        b0 = pl.multiple_of(base % nb, C)
        return out_hbm.at[tt].at[pl.ds(b0, C)]

    def fire_wb(ch, buf):
        pltpu.async_copy(rows[buf], out_slab(ch, buf), sw[buf])

    def wait_wb(buf):
        pltpu.make_async_copy(rows[buf],
                              out_hbm.at[0].at[pl.ds(0, C)], sw[buf]).wait()

    def pack(ch, buf):
        r = rows[buf]
        cbase = pl.multiple_of(ch * C, C)

        def body(gi, carry):
            pv = (tok_v[pl.ds(cbase + gi * LANES, LANES)] & 1).astype(
                jnp.float32)
            for j in range(LANES):
                m = gi * LANES + j
                pf = jnp.broadcast_to(pv[j], (LANES,))
                for c in range(EMB // LANES):
                    lo = r[m, pl.ds(c * LANES, LANES)]
                    hi = r[m, pl.ds(EMB + c * LANES, LANES)]
                    r[m, pl.ds(c * LANES, LANES)] = (
                        lo + pf * (hi - lo)) * SCALE
            return carry

        lax.fori_loop(0, C // LANES, body, 0)

    def run_chunk(ch, buf, fire_next, wait_prev_wb):
        if wait_prev_wb:
            wait_wb(1 - buf)
        if fire_next:
            fire_gathers(ch + 1, 1 - buf)
        wait_gathers(buf)
        pack(ch, buf)
        fire_wb(ch, buf)

    fire_gathers(0, 0)
    run_chunk(0, 0, fire_next=True, wait_prev_wb=False)

    def pair_body(i, carry):
        ch = 1 + 2 * i
        run_chunk(ch, 1, fire_next=True, wait_prev_wb=True)
        run_chunk(ch + 1, 0, fire_next=True, wait_prev_wb=True)
        return carry

    lax.fori_loop(0, (chunks - 2) // 2, pair_body, 0)

    run_chunk(chunks - 1, 1, fire_next=False, wait_prev_wb=True)
    wait_wb(1)


def kernel(tokens, table):
    b, t = tokens.shape
    n = b * t
    v = table.shape[0]
    tp = tokens.T.astype(jnp.int32).reshape(n)
    table2 = table.reshape(v // 2, 2 * EMB)

    mesh = plsc.VectorSubcoreMesh(core_axis_name="c", subcore_axis_name="s")
    run = functools.partial(
        pl.kernel,
        mesh=mesh,
        compiler_params=pltpu.CompilerParams(use_tc_tiling_on_sc=True),
        out_type=jax.ShapeDtypeStruct((t, b, 2 * EMB), jnp.float32),
        scratch_types=[
            pltpu.VMEM((n // NW,), jnp.int32),
            pltpu.VMEM((C,), jnp.int32),
            pltpu.VMEM((C,), jnp.int32),
            pltpu.VMEM((C, 2 * EMB), jnp.float32),
            pltpu.VMEM((C, 2 * EMB), jnp.float32),
            pltpu.SemaphoreType.DMA,
            pltpu.SemaphoreType.DMA,
            pltpu.SemaphoreType.DMA,
            pltpu.SemaphoreType.DMA,
        ],
    )(_emb_body)
    out = run(tp, table2)
    return out[:, :, :EMB].transpose(1, 0, 2)


# speedup vs baseline: 1.3692x; 1.2084x over previous
import functools
import math

import jax
import jax.numpy as jnp
from jax import lax
from jax.experimental import pallas as pl
from jax.experimental.pallas import tpu as pltpu
from jax.experimental.pallas import tpu_sc as plsc

EMB = 64
LANES = 16
IDXW = 128
K = 1
C = K * IDXW
NC = 2
NS = 16
NW = NC * NS
SCALE = math.sqrt(EMB)


def _emb_body(tok_hbm, table_hbm, out_hbm,
              tok_v, gidx0, gidx1, rows0, rows1, st0, st1,
              sg0, sg1, sw0, sw1):
    wid = lax.axis_index("s") * NC + lax.axis_index("c")
    n = tok_hbm.shape[0]
    tok_w = n // NW
    chunks = tok_w // C
    nb = out_hbm.shape[1]
    tok0 = pl.multiple_of(wid * tok_w, 1024)

    rows = (rows0, rows1)
    stage = (st0, st1)
    gidx = (gidx0, gidx1)
    sg = (sg0, sg1)
    sw = (sw0, sw1)

    pltpu.sync_copy(tok_hbm.at[pl.ds(tok0, tok_w)], tok_v)

    def fire_gathers(ch, buf):
        base = pl.multiple_of(ch * C, C)
        g = gidx[buf]
        for i in range(C // LANES):
            g[pl.ds(i * LANES, LANES)] = jnp.right_shift(
                tok_v[pl.ds(base + i * LANES, LANES)], 1)
        for j in range(K):
            pltpu.async_copy(
                table_hbm.at[g.at[pl.ds(j * IDXW, IDXW)]],
                rows[buf].at[pl.ds(j * IDXW, IDXW)],
                sg[buf],
            )

    def wait_gathers(buf):
        for _ in range(K):
            pltpu.make_async_copy(table_hbm.at[pl.ds(0, IDXW)],
                                  rows[buf].at[pl.ds(0, IDXW)], sg[buf]).wait()

    def out_slab(ch, buf):
        base = tok0 + ch * C
        tt = base // nb
        b0 = pl.multiple_of(base % nb, C)
        return out_hbm.at[tt].at[pl.ds(b0, C)]

    def fire_wb(ch, buf):
        pltpu.async_copy(stage[buf], out_slab(ch, buf), sw[buf])

    def wait_wb(buf):
        pltpu.make_async_copy(stage[buf],
                              out_hbm.at[0].at[pl.ds(0, C)], sw[buf]).wait()

    def pack(ch, buf):
        r = rows[buf]
        cbase = pl.multiple_of(ch * C, C)

        st = stage[buf]

        def body(gi, carry):
            pv = (tok_v[pl.ds(cbase + gi * LANES, LANES)] & 1).astype(
                jnp.float32)
            for j in range(LANES):
                m = gi * LANES + j
                pf = jnp.broadcast_to(pv[j], (LANES,))
                for c in range(EMB // LANES):
                    lo = r[m, pl.ds(c * LANES, LANES)]
                    hi = r[m, pl.ds(EMB + c * LANES, LANES)]
                    st[m, pl.ds(c * LANES, LANES)] = (
                        lo + pf * (hi - lo)) * SCALE
            return carry

        lax.fori_loop(0, C // LANES, body, 0)

    def run_chunk(ch, buf, fire_next, wait_prev_wb):
        if wait_prev_wb:
            wait_wb(1 - buf)
        if fire_next:
            fire_gathers(ch + 1, 1 - buf)
        wait_gathers(buf)
        pack(ch, buf)
        fire_wb(ch, buf)

    fire_gathers(0, 0)
    run_chunk(0, 0, fire_next=True, wait_prev_wb=False)

    def pair_body(i, carry):
        ch = 1 + 2 * i
        run_chunk(ch, 1, fire_next=True, wait_prev_wb=True)
        run_chunk(ch + 1, 0, fire_next=True, wait_prev_wb=True)
        return carry

    lax.fori_loop(0, (chunks - 2) // 2, pair_body, 0)

    run_chunk(chunks - 1, 1, fire_next=False, wait_prev_wb=True)
    wait_wb(1)


def kernel(tokens, table):
    b, t = tokens.shape
    n = b * t
    v = table.shape[0]
    tp = tokens.T.astype(jnp.int32).reshape(n)
    table2 = table.reshape(v // 2, 2 * EMB)

    mesh = plsc.VectorSubcoreMesh(core_axis_name="c", subcore_axis_name="s")
    run = functools.partial(
        pl.kernel,
        mesh=mesh,
        compiler_params=pltpu.CompilerParams(use_tc_tiling_on_sc=True),
        out_type=jax.ShapeDtypeStruct((t, b, EMB), jnp.float32),
        scratch_types=[
            pltpu.VMEM((n // NW,), jnp.int32),
            pltpu.VMEM((C,), jnp.int32),
            pltpu.VMEM((C,), jnp.int32),
            pltpu.VMEM((C, 2 * EMB), jnp.float32),
            pltpu.VMEM((C, 2 * EMB), jnp.float32),
            pltpu.VMEM((C, EMB), jnp.float32),
            pltpu.VMEM((C, EMB), jnp.float32),
            pltpu.SemaphoreType.DMA,
            pltpu.SemaphoreType.DMA,
            pltpu.SemaphoreType.DMA,
            pltpu.SemaphoreType.DMA,
        ],
    )(_emb_body)
    out = run(tp, table2)
    return out.transpose(1, 0, 2)
